# baseline (speedup 1.0000x reference)
import functools

import jax
import jax.numpy as jnp
from jax import lax
from jax.experimental import pallas as pl
from jax.experimental.pallas import tpu as pltpu
from jax.experimental.pallas import tpu_sc as plsc

_TCB = 3
_BS = 1024
_CH = 32
_NW = 32


def _tc_body(lens_ref, x_ref, mask_ref, o_ref):
    b = pl.program_id(0)
    s = pl.program_id(1)
    first = s * _BS
    rows = first + jax.lax.broadcasted_iota(jnp.int32, (_BS, 1), 0)
    pad = rows >= lens_ref[b]
    o_ref[0] = jnp.where(pad, mask_ref[0][None, :], x_ref[0])


def _tc_x_map(b, s, lens_ref):
    last_live = jnp.maximum(jax.lax.div(lens_ref[b] + (_BS - 1), _BS) - 1, 0)
    return (b, jnp.minimum(s, last_live), 0)


def _tc_call(x, lens_i, mask2):
    B, S, F = x.shape
    grid_spec = pltpu.PrefetchScalarGridSpec(
        num_scalar_prefetch=1,
        grid=(B, S // _BS),
        in_specs=[
            pl.BlockSpec((1, _BS, F), _tc_x_map),
            pl.BlockSpec((1, F), lambda b, s, lens_ref: (0, 0)),
        ],
        out_specs=pl.BlockSpec((1, _BS, F), lambda b, s, lens_ref: (b, s, 0)),
    )
    return pl.pallas_call(
        _tc_body,
        grid_spec=grid_spec,
        out_shape=jax.ShapeDtypeStruct((B, S, F), x.dtype),
    )(lens_i, x, mask2)


def _make_sc_kernel(B, S, F):
    mesh = plsc.VectorSubcoreMesh(core_axis_name="c", subcore_axis_name="s")
    wpb = _NW // B
    n_local = S // _CH // wpb
    stride = _CH * wpb

    @functools.partial(
        pl.kernel,
        out_type=jax.ShapeDtypeStruct((B, S, F), jnp.float32),
        mesh=mesh,
        scratch_types=[
            pltpu.VMEM((_CH, F), jnp.float32),
            pltpu.VMEM((_CH, F), jnp.float32),
            pltpu.VMEM((_CH, F), jnp.float32),
            pltpu.VMEM((1, 16), jnp.int32),
            pltpu.SemaphoreType.DMA,
            pltpu.SemaphoreType.DMA,
            pltpu.SemaphoreType.DMA,
            pltpu.SemaphoreType.DMA,
            pltpu.SemaphoreType.DMA,
        ],
    )
    def sc_kernel(x_hbm, params_hbm, fill_hbm, out_hbm,
                  fillbuf, stage0, stage1, pbuf, sf, si0, si1, so0, so1):
        wid = lax.axis_index("c") * 16 + lax.axis_index("s")
        b = wid // wpb
        seg = wid % wpb
        pltpu.sync_copy(fill_hbm, fillbuf)
        pltpu.sync_copy(params_hbm.at[wid], pbuf)
        v = pbuf[0, :]
        n_copy = v[0]
        frac = v[1]
        owner = (frac > 0).astype(jnp.int32)

        def rowof(i):
            return seg * _CH + i * stride

        def cin(i, stg, sem):
            return pltpu.make_async_copy(
                x_hbm.at[b, pl.ds(rowof(i), _CH)], stg, sem)

        def cout(i, stg, sem):
            return pltpu.make_async_copy(
                stg, out_hbm.at[b, pl.ds(rowof(i), _CH)], sem)

        def fdma(i):
            return pltpu.make_async_copy(
                fillbuf, out_hbm.at[b, pl.ds(rowof(i), _CH)], sf)

        n_fill0 = n_copy + owner

        def fire_fill(i, c):
            fdma(i).start()
            return c

        lax.fori_loop(n_fill0, n_local, fire_fill, 0)

        @pl.when(n_copy > 0)
        def _():
            cin(0, stage0, si0).start()

        @pl.when(n_copy > 1)
        def _():
            cin(1, stage1, si1).start()

        def copy_body(i, c):
            even = i % 2 == 0

            @pl.when(even)
            def _():
                cin(i, stage0, si0).wait()
                cout(i, stage0, so0).start()

                @pl.when(i + 2 < n_copy)
                def _():
                    cout(i, stage0, so0).wait()
                    cin(i + 2, stage0, si0).start()

            @pl.when(jnp.logical_not(even))
            def _():
                cin(i, stage1, si1).wait()
                cout(i, stage1, so1).start()

                @pl.when(i + 2 < n_copy)
                def _():
                    cout(i, stage1, so1).wait()
                    cin(i + 2, stage1, si1).start()

            return c

        lax.fori_loop(0, n_copy, copy_body, 0)

        def drain(i):
            @pl.when(i % 2 == 0)
            def _():
                cout(i, stage0, so0).wait()

            @pl.when(i % 2 == 1)
            def _():
                cout(i, stage1, so1).wait()

        @pl.when(n_copy > 1)
        def _():
            drain(n_copy - 2)

        @pl.when(n_copy > 0)
        def _():
            drain(n_copy - 1)

        @pl.when(frac > 0)
        def _():
            row = rowof(n_copy)
            pltpu.sync_copy(x_hbm.at[b, pl.ds(row, _CH)], stage0)

            def patch(r, c):
                for j in range(F // 16):
                    stage0[r, pl.ds(j * 16, 16)] = fillbuf[0, pl.ds(j * 16, 16)]
                return c

            lax.fori_loop(frac, _CH, patch, 0)
            pltpu.sync_copy(stage0, out_hbm.at[b, pl.ds(row, _CH)])

        def drain_fill(i, c):
            fdma(i).wait()
            return c

        lax.fori_loop(n_fill0, n_local, drain_fill, 0)

    return sc_kernel


def _sc_call(x, lens_i, output_mask):
    B, S, F = x.shape
    wpb = _NW // B
    n_local = S // _CH // wpb
    wids = jnp.arange(_NW, dtype=jnp.int32)
    cut = jnp.clip(lens_i[wids // wpb], 0, S)
    gc = cut // _CH
    frac_b = cut - gc * _CH
    seg = wids % wpb
    n_copy = jnp.clip((gc - seg + (wpb - 1)) // wpb, 0, n_local)
    owner = (frac_b > 0) & (gc % wpb == seg)
    frac = jnp.where(owner, frac_b, 0)
    params = jnp.stack([n_copy, frac], axis=1)
    params = jnp.pad(params, ((0, 0), (0, 14)))[:, None, :]
    fill = jnp.broadcast_to(output_mask[None, :], (_CH, F))
    return _make_sc_kernel(B, S, F)(x, params, fill)


def kernel(x, lens, output_mask):
    B, S, F = x.shape
    lens_i = lens.astype(jnp.int32)
    mask2 = output_mask.reshape(1, F)
    out_tc = _tc_call(x[:_TCB], lens_i[:_TCB], mask2)
    out_sc = _sc_call(x[_TCB:], lens_i[_TCB:], output_mask)
    return jnp.concatenate([out_tc, out_sc], axis=0)


# speedup vs baseline: 2.5663x; 2.5663x over previous
import functools

import jax
import jax.numpy as jnp
from jax import lax
from jax.experimental import pallas as pl
from jax.experimental.pallas import tpu as pltpu
from jax.experimental.pallas import tpu_sc as plsc

_CC = 32
_FC = 64
_WPB = 8


def _make_sc_kernel(B, S, F, NW):
    mesh = plsc.VectorSubcoreMesh(core_axis_name="c", subcore_axis_name="s")
    n_fill_local = S // _FC // _WPB

    @functools.partial(
        pl.kernel,
        out_type=jax.ShapeDtypeStruct((B, S, F), jnp.float32),
        mesh=mesh,
        scratch_types=[
            pltpu.VMEM((_FC, F), jnp.float32),
            pltpu.VMEM((_CC, F), jnp.float32),
            pltpu.VMEM((1, 16), jnp.int32),
            pltpu.SemaphoreType.DMA,
            pltpu.SemaphoreType.DMA,
        ],
    )
    def sc_kernel(x_hbm, params_hbm, fill_hbm, out_hbm,
                  fillbuf, stage, pbuf, sf, se):
        wid = lax.axis_index("c") * 16 + lax.axis_index("s")
        b = wid // _WPB
        seg = wid % _WPB
        pltpu.sync_copy(fill_hbm, fillbuf)
        pltpu.sync_copy(params_hbm.at[wid], pbuf)
        v = pbuf[0, :]
        n_copy = v[0]
        frac = v[1]
        e32 = v[2]
        nf_start = v[3]

        def crow(i):
            return seg * _CC + i * (_CC * _WPB)

        def frow(j):
            return seg * _FC + j * (_FC * _WPB)

        def fdma(j):
            return pltpu.make_async_copy(
                fillbuf, out_hbm.at[b, pl.ds(frow(j), _FC)], sf)

        def edma():
            return pltpu.make_async_copy(
                fillbuf.at[pl.ds(0, _CC)],
                out_hbm.at[b, pl.ds(crow(n_copy), _CC)], se)

        def fire_fill(j, c):
            fdma(j).start()
            return c

        lax.fori_loop(nf_start, n_fill_local, fire_fill, 0)

        @pl.when(e32 > 0)
        def _():
            edma().start()

        def copy_body(i, c):
            row = crow(i)
            pltpu.sync_copy(x_hbm.at[b, pl.ds(row, _CC)], stage)
            pltpu.sync_copy(stage, out_hbm.at[b, pl.ds(row, _CC)])
            return c

        lax.fori_loop(0, n_copy, copy_body, 0)

        @pl.when(frac > 0)
        def _():
            row = crow(n_copy)
            pltpu.sync_copy(x_hbm.at[b, pl.ds(row, _CC)], stage)

            def patch(r, c):
                for j in range(F // 16):
                    stage[r, pl.ds(j * 16, 16)] = fillbuf[0, pl.ds(j * 16, 16)]
                return c

            lax.fori_loop(frac, _CC, patch, 0)
            pltpu.sync_copy(stage, out_hbm.at[b, pl.ds(row, _CC)])

        def drain_fill(j, c):
            fdma(j).wait()
            return c

        lax.fori_loop(nf_start, n_fill_local, drain_fill, 0)

        @pl.when(e32 > 0)
        def _():
            edma().wait()

    return sc_kernel


def kernel(x, lens, output_mask):
    B, S, F = x.shape
    NW = B * _WPB
    n_copy_local = S // _CC // _WPB
    n_fill_local = S // _FC // _WPB
    lens_i = lens.astype(jnp.int32)
    wids = jnp.arange(NW, dtype=jnp.int32)
    cut = jnp.clip(lens_i[wids // _WPB], 0, S)
    gc = cut // _CC
    frac_b = cut - gc * _CC
    g_left = gc + (frac_b > 0)
    m64k = (cut + _FC - 1) // _FC
    r32 = g_left * _CC < m64k * _FC
    seg = wids % _WPB
    n_copy = jnp.clip((gc - seg + (_WPB - 1)) // _WPB, 0, n_copy_local)
    owner_m = (frac_b > 0) & (gc % _WPB == seg)
    frac = jnp.where(owner_m, frac_b, 0)
    e32 = (r32 & (g_left % _WPB == seg)).astype(jnp.int32)
    nf_start = jnp.clip((m64k - seg + (_WPB - 1)) // _WPB, 0, n_fill_local)
    params = jnp.stack([n_copy, frac, e32, nf_start], axis=1)
    params = jnp.pad(params, ((0, 0), (0, 12)))[:, None, :]
    fill = jnp.broadcast_to(output_mask[None, :], (_FC, F))
    return _make_sc_kernel(B, S, F, NW)(x, params, fill)


# speedup vs baseline: 2.7772x; 1.0821x over previous
import functools

import jax
import jax.numpy as jnp
from jax import lax
from jax.experimental import pallas as pl
from jax.experimental.pallas import tpu as pltpu
from jax.experimental.pallas import tpu_sc as plsc

_CH = 32
_WPB = 8


def _make_sc_kernel(B, S, F, NW):
    mesh = plsc.VectorSubcoreMesh(core_axis_name="c", subcore_axis_name="s")
    n_local = S // _CH // _WPB
    stride = _CH * _WPB

    @functools.partial(
        pl.kernel,
        out_type=jax.ShapeDtypeStruct((B, S, F), jnp.float32),
        mesh=mesh,
        scratch_types=[
            pltpu.VMEM((_CH, F), jnp.float32),
            pltpu.VMEM((_CH, F), jnp.float32),
            pltpu.VMEM((_CH, F), jnp.float32),
            pltpu.VMEM((1, 16), jnp.int32),
            pltpu.VMEM_SHARED((_CH, F), jnp.float32),
            pltpu.SemaphoreType.DMA,
            pltpu.SemaphoreType.DMA,
            pltpu.SemaphoreType.DMA,
            pltpu.SemaphoreType.DMA,
            pltpu.SemaphoreType.DMA,
            pltpu.SemaphoreType.DMA,
        ],
    )
    def sc_kernel(x_hbm, params_hbm, fill_hbm, out_hbm,
                  fillbuf, stage0, stage1, pbuf, sbuf,
                  sf, sb, si0, si1, so0, so1):
        wid = lax.axis_index("c") * 16 + lax.axis_index("s")
        sid = lax.axis_index("s")
        b = wid // _WPB
        seg = wid % _WPB

        @pl.when(sid == 0)
        def _():
            pltpu.sync_copy(fill_hbm, sbuf)

        pltpu.sync_copy(fill_hbm, fillbuf)
        pltpu.sync_copy(params_hbm.at[wid], pbuf)
        v = pbuf[0, :]
        n_copy = v[0]
        frac = v[1]
        owner = (frac > 0).astype(jnp.int32)

        plsc.subcore_barrier()

        def rowof(i):
            return seg * _CH + i * stride

        def cin(i, stg, sem):
            return pltpu.make_async_copy(
                x_hbm.at[b, pl.ds(rowof(i), _CH)], stg, sem)

        def cout(i, stg, sem):
            return pltpu.make_async_copy(
                stg, out_hbm.at[b, pl.ds(rowof(i), _CH)], sem)

        def fdma(i):
            return pltpu.make_async_copy(
                fillbuf, out_hbm.at[b, pl.ds(rowof(i), _CH)], sf)

        def bdma(i):
            return pltpu.make_async_copy(
                sbuf, out_hbm.at[b, pl.ds(rowof(i), _CH)], sb)

        n_fill0 = n_copy + owner

        def fire_fill(i, c):
            @pl.when(i % 3 == 2)
            def _():
                bdma(i).start()

            @pl.when(i % 3 != 2)
            def _():
                fdma(i).start()

            return c

        lax.fori_loop(n_fill0, n_local, fire_fill, 0)

        @pl.when(n_copy > 0)
        def _():
            cin(0, stage0, si0).start()

        @pl.when(n_copy > 1)
        def _():
            cin(1, stage1, si1).start()

        def copy_body(i, c):
            even = i % 2 == 0

            @pl.when(even)
            def _():
                cin(i, stage0, si0).wait()
                cout(i, stage0, so0).start()

                @pl.when(i + 2 < n_copy)
                def _():
                    cout(i, stage0, so0).wait()
                    cin(i + 2, stage0, si0).start()

            @pl.when(jnp.logical_not(even))
            def _():
                cin(i, stage1, si1).wait()
                cout(i, stage1, so1).start()

                @pl.when(i + 2 < n_copy)
                def _():
                    cout(i, stage1, so1).wait()
                    cin(i + 2, stage1, si1).start()

            return c

        lax.fori_loop(0, n_copy, copy_body, 0)

        def drain(i):
            @pl.when(i % 2 == 0)
            def _():
                cout(i, stage0, so0).wait()

            @pl.when(i % 2 == 1)
            def _():
                cout(i, stage1, so1).wait()

        @pl.when(n_copy > 1)
        def _():
            drain(n_copy - 2)

        @pl.when(n_copy > 0)
        def _():
            drain(n_copy - 1)

        @pl.when(frac > 0)
        def _():
            row = rowof(n_copy)
            pltpu.sync_copy(x_hbm.at[b, pl.ds(row, _CH)], stage0)

            def patch(r, c):
                for j in range(F // 16):
                    stage0[r, pl.ds(j * 16, 16)] = fillbuf[0, pl.ds(j * 16, 16)]
                return c

            lax.fori_loop(frac, _CH, patch, 0)
            pltpu.sync_copy(stage0, out_hbm.at[b, pl.ds(row, _CH)])

        def drain_fill(i, c):
            @pl.when(i % 3 == 2)
            def _():
                bdma(i).wait()

            @pl.when(i % 3 != 2)
            def _():
                fdma(i).wait()

            return c

        lax.fori_loop(n_fill0, n_local, drain_fill, 0)

    return sc_kernel


def kernel(x, lens, output_mask):
    B, S, F = x.shape
    NW = B * _WPB
    n_local = S // _CH // _WPB
    lens_i = lens.astype(jnp.int32)
    wids = jnp.arange(NW, dtype=jnp.int32)
    cut = jnp.clip(lens_i[wids // _WPB], 0, S)
    gc = cut // _CH
    frac_b = cut - gc * _CH
    seg = wids % _WPB
    n_copy = jnp.clip((gc - seg + (_WPB - 1)) // _WPB, 0, n_local)
    owner = (frac_b > 0) & (gc % _WPB == seg)
    frac = jnp.where(owner, frac_b, 0)
    params = jnp.stack([n_copy, frac], axis=1)
    params = jnp.pad(params, ((0, 0), (0, 14)))[:, None, :]
    fill = jnp.broadcast_to(output_mask[None, :], (_CH, F))
    return _make_sc_kernel(B, S, F, NW)(x, params, fill)


# speedup vs baseline: 2.9728x; 1.0705x over previous
import functools

import jax
import jax.numpy as jnp
from jax import lax
from jax.experimental import pallas as pl
from jax.experimental.pallas import tpu as pltpu
from jax.experimental.pallas import tpu_sc as plsc

_CH = 32
_WPB = 8


def _make_sc_kernel(B, S, F, NW):
    mesh = plsc.VectorSubcoreMesh(core_axis_name="c", subcore_axis_name="s")
    n_local = S // _CH // _WPB
    stride = _CH * _WPB

    @functools.partial(
        pl.kernel,
        out_type=jax.ShapeDtypeStruct((B, S, F), jnp.float32),
        mesh=mesh,
        scratch_types=[
            pltpu.VMEM((_CH, F), jnp.float32),
            pltpu.VMEM((_CH, F), jnp.float32),
            pltpu.VMEM((_CH, F), jnp.float32),
            pltpu.VMEM((1, 16), jnp.int32),
            pltpu.SemaphoreType.DMA,
            pltpu.SemaphoreType.DMA,
            pltpu.SemaphoreType.DMA,
            pltpu.SemaphoreType.DMA,
            pltpu.SemaphoreType.DMA,
        ],
    )
    def sc_kernel(x_hbm, params_hbm, fill_hbm, out_hbm,
                  fillbuf, stage0, stage1, pbuf,
                  sf, si0, si1, so0, so1):
        wid = lax.axis_index("c") * 16 + lax.axis_index("s")
        b = wid // _WPB
        seg = wid % _WPB

        pltpu.make_async_copy(fill_hbm, fillbuf, si0).start()
        pltpu.make_async_copy(params_hbm.at[wid], pbuf, si1).start()
        pltpu.make_async_copy(params_hbm.at[wid], pbuf, si1).wait()
        pltpu.make_async_copy(fill_hbm, fillbuf, si0).wait()
        v = pbuf[0, :]
        n_copy = v[0]
        frac = v[1]
        owner = (frac > 0).astype(jnp.int32)

        def rowof(i):
            return seg * _CH + i * stride

        def cin(i, stg, sem):
            return pltpu.make_async_copy(
                x_hbm.at[b, pl.ds(rowof(i), _CH)], stg, sem)

        def cout(i, stg, sem):
            return pltpu.make_async_copy(
                stg, out_hbm.at[b, pl.ds(rowof(i), _CH)], sem)

        def fdma(i):
            return pltpu.make_async_copy(
                fillbuf, out_hbm.at[b, pl.ds(rowof(i), _CH)], sf)

        n_fill0 = n_copy + owner

        def fire_fill(i, c):
            fdma(i).start()
            return c

        lax.fori_loop(n_fill0, n_local, fire_fill, 0)

        @pl.when(n_copy > 0)
        def _():
            cin(0, stage0, si0).start()

        @pl.when(n_copy > 1)
        def _():
            cin(1, stage1, si1).start()

        def copy_body(i, c):
            even = i % 2 == 0

            @pl.when(even)
            def _():
                cin(i, stage0, si0).wait()
                cout(i, stage0, so0).start()

                @pl.when(i + 2 < n_copy)
                def _():
                    cout(i, stage0, so0).wait()
                    cin(i + 2, stage0, si0).start()

            @pl.when(jnp.logical_not(even))
            def _():
                cin(i, stage1, si1).wait()
                cout(i, stage1, so1).start()

                @pl.when(i + 2 < n_copy)
                def _():
                    cout(i, stage1, so1).wait()
                    cin(i + 2, stage1, si1).start()

            return c

        lax.fori_loop(0, n_copy, copy_body, 0)

        def drain(i):
            @pl.when(i % 2 == 0)
            def _():
                cout(i, stage0, so0).wait()

            @pl.when(i % 2 == 1)
            def _():
                cout(i, stage1, so1).wait()

        @pl.when(n_copy > 1)
        def _():
            drain(n_copy - 2)

        @pl.when(n_copy > 0)
        def _():
            drain(n_copy - 1)

        @pl.when(frac > 0)
        def _():
            row = rowof(n_copy)
            pltpu.sync_copy(x_hbm.at[b, pl.ds(row, _CH)], stage0)

            def patch(r, c):
                for j in range(F // 16):
                    stage0[r, pl.ds(j * 16, 16)] = fillbuf[0, pl.ds(j * 16, 16)]
                return c

            lax.fori_loop(frac, _CH, patch, 0)
            pltpu.sync_copy(stage0, out_hbm.at[b, pl.ds(row, _CH)])

        def drain_fill(i, c):
            fdma(i).wait()
            return c

        lax.fori_loop(n_fill0, n_local, drain_fill, 0)

    return sc_kernel


def kernel(x, lens, output_mask):
    B, S, F = x.shape
    NW = B * _WPB
    n_local = S // _CH // _WPB
    lens_i = lens.astype(jnp.int32)
    wids = jnp.arange(NW, dtype=jnp.int32)
    cut = jnp.clip(lens_i[wids // _WPB], 0, S)
    gc = cut // _CH
    frac_b = cut - gc * _CH
    seg = wids % _WPB
    n_copy = jnp.clip((gc - seg + (_WPB - 1)) // _WPB, 0, n_local)
    owner = (frac_b > 0) & (gc % _WPB == seg)
    frac = jnp.where(owner, frac_b, 0)
    params = jnp.stack([n_copy, frac], axis=1)
    params = jnp.pad(params, ((0, 0), (0, 14)))[:, None, :]
    fill = jnp.broadcast_to(output_mask[None, :], (_CH, F))
    return _make_sc_kernel(B, S, F, NW)(x, params, fill)
